# Initial kernel scaffold; baseline (speedup 1.0000x reference)
import jax
import jax.numpy as jnp
from jax.experimental import pallas as pl


def kernel(positions, shifts, node_attrs, edge_index, batch, params):
    raise NotImplementedError("write your pallas kernel here")


# speedup vs baseline: 7.0085x; 7.0085x over previous
import functools

import jax
import jax.numpy as jnp
from jax import lax
from jax.experimental import pallas as pl
from jax.experimental.pallas import tpu as pltpu

F32 = jnp.float32

N_BASES = 6
N_POLY = 6
CUTOFF = 5.0
N_SCAL = 16
N_VEC = 8
N_SEDGE = 16
N_GRAPHS = 64

BE = 2000
BN = 2000


def _sigmoid(x):
    return jax.nn.sigmoid(x)


def _silu(x):
    return x * jax.nn.sigmoid(x)


def _ln16(s, g, b, eps=1e-5):
    mu = jnp.mean(s, axis=-1, keepdims=True)
    var = jnp.mean((s - mu) * (s - mu), axis=-1, keepdims=True)
    return (s - mu) * jax.lax.rsqrt(var + eps) * g + b


def _edge_feat_body(ps_ref, pd_ref, sh_ref, lng_ref, lnb_ref,
                    wsw_ref, wsb_ref, whv_ref, wvv_ref, wsvw_ref, wsvb_ref,
                    es_ref, evt_ref):
    v = pd_ref[...] - ps_ref[...] + sh_ref[...]
    l2 = jnp.sum(v * v, axis=1, keepdims=True)
    lengths = jnp.sqrt(l2)
    unit = v / (lengths + 1e-9)
    w_row = jnp.pi * (1.0 + lax.broadcasted_iota(
        jnp.int32, (1, N_BASES), 1).astype(F32)) / CUTOFF
    rb = jnp.sqrt(2.0 / CUTOFF) * jnp.sin(lengths * w_row) / lengths
    x = lengths / CUTOFF
    p = float(N_POLY)
    x6 = (x * x * x) ** 2
    env = (1.0 - (p + 1.0) * (p + 2.0) / 2.0 * x6
           + p * (p + 2.0) * x6 * x
           - p * (p + 1.0) / 2.0 * x6 * x * x)
    env = env * (x < 1.0).astype(F32)
    e_s = rb * env
    es_n = _ln16(e_s, lng_ref[...], lnb_ref[...])
    nv2 = jnp.maximum(jnp.sum(unit * unit, axis=1, keepdims=True), 1e-8)
    ev_n = unit * jax.lax.rsqrt(nv2)
    vh = ev_n * whv_ref[...]
    vn1 = jnp.sqrt(jnp.maximum(jnp.sum(vh * vh, axis=1, keepdims=True), 1e-8))
    s16 = (jnp.dot(es_n, wsw_ref[0:N_BASES, :], preferred_element_type=F32)
           + vn1 * wsw_ref[N_BASES:N_BASES + 1, :]
           + wsb_ref[...])
    vout = vh * wvv_ref[...]
    gate = jnp.dot(s16, wsvw_ref[...], preferred_element_type=F32) + wsvb_ref[...]
    es_ref[...] = s16
    evt_ref[...] = vout * _sigmoid(gate)


def _edge_feats(ps, pd, sh, pe):
    E = ps.shape[0]
    g = pe['gvp']
    eb = lambda i: (i, 0)
    wb = lambda a: pl.BlockSpec(a.shape, lambda i: (0,) * a.ndim)
    args = [pe['ln_g'].reshape(1, -1), pe['ln_b'].reshape(1, -1),
            g['ws_w'], g['ws_b'].reshape(1, -1),
            g['wh'], g['wv'], g['wsv_w'], g['wsv_b'].reshape(1, -1)]
    return pl.pallas_call(
        _edge_feat_body,
        grid=(E // BE,),
        in_specs=[pl.BlockSpec((BE, 3), eb), pl.BlockSpec((BE, 3), eb),
                  pl.BlockSpec((BE, 3), eb)] + [wb(a) for a in args],
        out_specs=[pl.BlockSpec((BE, N_SCAL), eb), pl.BlockSpec((BE, 3), eb)],
        out_shape=[jax.ShapeDtypeStruct((E, N_SCAL), F32),
                   jax.ShapeDtypeStruct((E, 3), F32)],
    )(ps, pd, sh, *args)


def _node_init_body(na_ref, lng_ref, lnb_ref, wsw_ref, wsb_ref, hs_ref):
    s = _ln16(na_ref[...], lng_ref[...], lnb_ref[...])
    hs_ref[...] = jnp.dot(s, wsw_ref[...], preferred_element_type=F32) + wsb_ref[...]


def _node_init(node_attrs, pv):
    N, T = node_attrs.shape
    g = pv['gvp']
    nb = lambda i: (i, 0)
    wb = lambda a: pl.BlockSpec(a.shape, lambda i: (0,) * a.ndim)
    args = [pv['ln_g'].reshape(1, -1), pv['ln_b'].reshape(1, -1),
            g['ws_w'], g['ws_b'].reshape(1, -1)]
    hs = pl.pallas_call(
        _node_init_body,
        grid=(N // BN,),
        in_specs=[pl.BlockSpec((BN, T), nb)] + [wb(a) for a in args],
        out_specs=pl.BlockSpec((BN, N_SCAL), nb),
        out_shape=jax.ShapeDtypeStruct((N, N_SCAL), F32),
    )(node_attrs, *args)
    return hs


def _edge_msg_body(gss_ref, gsd_ref, gvs_ref, gvd_ref, es_ref, evt_ref,
                   whvs_ref, whe_ref, whvd_ref,
                   wsss_ref, wses_ref, wssd_ref, wsvn_ref, wsb1_ref,
                   wv1_ref, wsv1_ref, wsv1b_ref,
                   wh2_ref, ws2s_ref, ws2vn_ref, wsb2_ref,
                   wv2_ref, wsv2_ref, wsv2b_ref,
                   out_ref):
    evt = evt_ref[...]
    vh = []
    for x in range(3):
        vs_x = gvs_ref[:, 8 * x:8 * (x + 1)]
        vd_x = gvd_ref[:, 8 * x:8 * (x + 1)]
        e_x = evt[:, x:x + 1]
        vh.append(jnp.dot(vs_x, whvs_ref[...], preferred_element_type=F32)
                  + e_x * whe_ref[...]
                  + jnp.dot(vd_x, whvd_ref[...], preferred_element_type=F32))
    vn = jnp.sqrt(jnp.maximum(vh[0] * vh[0] + vh[1] * vh[1] + vh[2] * vh[2], 1e-8))
    s1 = (jnp.dot(gss_ref[...], wsss_ref[...], preferred_element_type=F32)
          + jnp.dot(es_ref[...], wses_ref[...], preferred_element_type=F32)
          + jnp.dot(gsd_ref[...], wssd_ref[...], preferred_element_type=F32)
          + jnp.dot(vn, wsvn_ref[...], preferred_element_type=F32)
          + wsb1_ref[...])
    sig1 = _sigmoid(jnp.dot(s1, wsv1_ref[...], preferred_element_type=F32) + wsv1b_ref[...])
    vo1 = [jnp.dot(vh[x], wv1_ref[...], preferred_element_type=F32) * sig1 for x in range(3)]
    s1a = _silu(s1)
    vh2 = [jnp.dot(vo1[x], wh2_ref[...], preferred_element_type=F32) for x in range(3)]
    vn2 = jnp.sqrt(jnp.maximum(vh2[0] * vh2[0] + vh2[1] * vh2[1] + vh2[2] * vh2[2], 1e-8))
    s2 = (jnp.dot(s1a, ws2s_ref[...], preferred_element_type=F32)
          + jnp.dot(vn2, ws2vn_ref[...], preferred_element_type=F32)
          + wsb2_ref[...])
    gate2 = _sigmoid(jnp.dot(s2, wsv2_ref[...], preferred_element_type=F32) + wsv2b_ref[...])
    vo2 = [jnp.dot(vh2[x], wv2_ref[...], preferred_element_type=F32) * gate2 for x in range(3)]
    out_ref[...] = jnp.concatenate([s2, vo2[0], vo2[1], vo2[2]], axis=1)


def _edge_msgs(gss, gsd, gvs, gvd, es, evt, lp):
    E = gss.shape[0]
    p1, p2 = lp['msg'][0], lp['msg'][1]
    wh = p1['wh']
    wsw = p1['ws_w']
    eb = lambda i: (i, 0)
    wb = lambda a: pl.BlockSpec(a.shape, lambda i: (0,) * a.ndim)
    args = [
        wh[0:N_VEC, :], wh[N_VEC:N_VEC + 1, :], wh[N_VEC + 1:, :],
        wsw[0:N_SCAL, :], wsw[N_SCAL:N_SCAL + N_SEDGE, :],
        wsw[N_SCAL + N_SEDGE:2 * N_SCAL + N_SEDGE, :], wsw[2 * N_SCAL + N_SEDGE:, :],
        p1['ws_b'].reshape(1, -1),
        p1['wv'], p1['wsv_w'], p1['wsv_b'].reshape(1, -1),
        p2['wh'], p2['ws_w'][0:N_SCAL, :], p2['ws_w'][N_SCAL:, :],
        p2['ws_b'].reshape(1, -1),
        p2['wv'], p2['wsv_w'], p2['wsv_b'].reshape(1, -1),
    ]
    return pl.pallas_call(
        _edge_msg_body,
        grid=(E // BE,),
        in_specs=[pl.BlockSpec((BE, N_SCAL), eb), pl.BlockSpec((BE, N_SCAL), eb),
                  pl.BlockSpec((BE, 3 * N_VEC), eb), pl.BlockSpec((BE, 3 * N_VEC), eb),
                  pl.BlockSpec((BE, N_SEDGE), eb), pl.BlockSpec((BE, 3), eb)]
                 + [wb(a) for a in args],
        out_specs=pl.BlockSpec((BE, N_SCAL + 3 * N_VEC), eb),
        out_shape=jax.ShapeDtypeStruct((E, N_SCAL + 3 * N_VEC), F32),
    )(gss, gsd, gvs, gvd, es, evt, *args)


def _node_upd_body(hs_ref, hv_ref, ssum_ref, vsum_ref, cnt_ref,
                   g1_ref, b1_ref, wh_ref, wss_ref, wsvn_ref, wsb_ref,
                   wv_ref, wsvw_ref, wsvb_ref, g2_ref, b2_ref,
                   hs2_ref, hv2_ref):
    inv = 1.0 / cnt_ref[...]
    s = hs_ref[...] + ssum_ref[...] * inv
    v = hv_ref[...] + vsum_ref[...] * inv
    vx = [v[:, 8 * x:8 * (x + 1)] for x in range(3)]
    vnc = jnp.maximum(vx[0] * vx[0] + vx[1] * vx[1] + vx[2] * vx[2], 1e-8)
    m = jnp.mean(vnc, axis=1, keepdims=True)
    rinv = jax.lax.rsqrt(m)
    vx = [a * rinv for a in vx]
    s = _ln16(s, g1_ref[...], b1_ref[...])
    vh = [jnp.dot(a, wh_ref[...], preferred_element_type=F32) for a in vx]
    vn = jnp.sqrt(jnp.maximum(vh[0] * vh[0] + vh[1] * vh[1] + vh[2] * vh[2], 1e-8))
    fs = (jnp.dot(s, wss_ref[...], preferred_element_type=F32)
          + jnp.dot(vn, wsvn_ref[...], preferred_element_type=F32) + wsb_ref[...])
    gate = _sigmoid(jnp.dot(fs, wsvw_ref[...], preferred_element_type=F32) + wsvb_ref[...])
    fv = [jnp.dot(vh[x], wv_ref[...], preferred_element_type=F32) * gate for x in range(3)]
    s2 = s + fs
    v2 = [vx[x] + fv[x] for x in range(3)]
    vnc2 = jnp.maximum(v2[0] * v2[0] + v2[1] * v2[1] + v2[2] * v2[2], 1e-8)
    m2 = jnp.mean(vnc2, axis=1, keepdims=True)
    rinv2 = jax.lax.rsqrt(m2)
    hs2_ref[...] = _ln16(s2, g2_ref[...], b2_ref[...])
    hv2_ref[...] = jnp.concatenate([a * rinv2 for a in v2], axis=1)


def _node_update(hs, hv, ssum, vsum, cnt2d, lp):
    N = hs.shape[0]
    ff = lp['ff'][0]
    nb = lambda i: (i, 0)
    wb = lambda a: pl.BlockSpec(a.shape, lambda i: (0,) * a.ndim)
    args = [lp['ln1_g'].reshape(1, -1), lp['ln1_b'].reshape(1, -1),
            ff['wh'], ff['ws_w'][0:N_SCAL, :], ff['ws_w'][N_SCAL:, :],
            ff['ws_b'].reshape(1, -1),
            ff['wv'], ff['wsv_w'], ff['wsv_b'].reshape(1, -1),
            lp['ln2_g'].reshape(1, -1), lp['ln2_b'].reshape(1, -1)]
    return pl.pallas_call(
        _node_upd_body,
        grid=(N // BN,),
        in_specs=[pl.BlockSpec((BN, N_SCAL), nb), pl.BlockSpec((BN, 3 * N_VEC), nb),
                  pl.BlockSpec((BN, N_SCAL), nb), pl.BlockSpec((BN, 3 * N_VEC), nb),
                  pl.BlockSpec((BN, 1), nb)] + [wb(a) for a in args],
        out_specs=[pl.BlockSpec((BN, N_SCAL), nb), pl.BlockSpec((BN, 3 * N_VEC), nb)],
        out_shape=[jax.ShapeDtypeStruct((N, N_SCAL), F32),
                   jax.ShapeDtypeStruct((N, 3 * N_VEC), F32)],
    )(hs, hv, ssum, vsum, cnt2d, *args)


def _out_body(hs_ref, hv_ref, batch_ref,
              g_ref, b_ref, wh_ref, wss_ref, wsvn_ref, wsb_ref,
              out_ref, acc_ref, cntacc_ref):
    i = pl.program_id(0)
    n = pl.num_programs(0)

    @pl.when(i == 0)
    def _init():
        acc_ref[...] = jnp.zeros_like(acc_ref)
        cntacc_ref[...] = jnp.zeros_like(cntacc_ref)

    v = hv_ref[...]
    vx = [v[:, 8 * x:8 * (x + 1)] for x in range(3)]
    vnc = jnp.maximum(vx[0] * vx[0] + vx[1] * vx[1] + vx[2] * vx[2], 1e-8)
    m = jnp.mean(vnc, axis=1, keepdims=True)
    rinv = jax.lax.rsqrt(m)
    vx = [a * rinv for a in vx]
    s = _ln16(hs_ref[...], g_ref[...], b_ref[...])
    vh = [jnp.dot(a, wh_ref[...], preferred_element_type=F32) for a in vx]
    vn = jnp.sqrt(jnp.maximum(vh[0] * vh[0] + vh[1] * vh[1] + vh[2] * vh[2], 1e-8))
    o = (jnp.dot(s, wss_ref[...], preferred_element_type=F32)
         + jnp.dot(vn, wsvn_ref[...], preferred_element_type=F32) + wsb_ref[...])
    o = _silu(o)
    gi = lax.broadcasted_iota(jnp.int32, (BN, N_GRAPHS), 1)
    oh = (batch_ref[...] == gi).astype(F32)
    acc_ref[...] += lax.dot_general(oh, o, (((0,), (0,)), ((), ())),
                                    preferred_element_type=F32)
    cntacc_ref[...] += jnp.sum(oh, axis=0, keepdims=True)

    @pl.when(i == n - 1)
    def _fin():
        out_ref[...] = acc_ref[...] / jnp.maximum(
            cntacc_ref[...].reshape(N_GRAPHS, 1), 1.0)


def _readout(hs, hv, batch2d, po):
    N = hs.shape[0]
    g = po['gvp']
    n_out = g['ws_w'].shape[1]
    nb = lambda i: (i, 0)
    wb = lambda a: pl.BlockSpec(a.shape, lambda i: (0,) * a.ndim)
    args = [po['ln_g'].reshape(1, -1), po['ln_b'].reshape(1, -1),
            g['wh'], g['ws_w'][0:N_SCAL, :], g['ws_w'][N_SCAL:, :],
            g['ws_b'].reshape(1, -1)]
    return pl.pallas_call(
        _out_body,
        grid=(N // BN,),
        in_specs=[pl.BlockSpec((BN, N_SCAL), nb), pl.BlockSpec((BN, 3 * N_VEC), nb),
                  pl.BlockSpec((BN, 1), nb)] + [wb(a) for a in args],
        out_specs=pl.BlockSpec((N_GRAPHS, n_out), lambda i: (0, 0)),
        out_shape=jax.ShapeDtypeStruct((N_GRAPHS, n_out), F32),
        scratch_shapes=[pltpu.VMEM((N_GRAPHS, n_out), F32),
                        pltpu.VMEM((1, N_GRAPHS), F32)],
    )(hs, hv, batch2d, *args)


def kernel(positions, shifts, node_attrs, edge_index, batch, params):
    src, dst = edge_index[0], edge_index[1]
    N = positions.shape[0]

    ps = positions[src]
    pd = positions[dst]
    es, evt = _edge_feats(ps, pd, shifts, params['W_e'])

    hs = _node_init(node_attrs, params['W_v'])
    hv = jnp.zeros((N, 3 * N_VEC), F32)

    ones = jnp.ones((dst.shape[0],), F32)
    cnt = jnp.maximum(jax.ops.segment_sum(ones, dst, num_segments=N), 1.0)
    cnt2d = cnt.reshape(N, 1)

    for lp in params['layers']:
        gss = hs[src]
        gsd = hs[dst]
        gvs = hv[src]
        gvd = hv[dst]
        msg = _edge_msgs(gss, gsd, gvs, gvd, es, evt, lp)
        sums = jax.ops.segment_sum(msg, dst, num_segments=N)
        hs, hv = _node_update(hs, hv, sums[:, :N_SCAL], sums[:, N_SCAL:], cnt2d, lp)

    batch2d = batch.reshape(N, 1)
    return _readout(hs, hv, batch2d, params['W_out'])
